# baseline (speedup 1.0000x reference)
import functools

import jax
import jax.numpy as jnp
from jax import lax
from jax.experimental import pallas as pl
from jax.experimental.pallas import tpu as pltpu
from jax.experimental.pallas import tpu_sc as plsc

N = 10000
D = 128
L = 3
CB = 16
NC = 2
NS = 16
NW = NC * NS
EB = 128
NBLK0 = 40
NBLK1 = 120
NBT = NBLK0 + NBLK1
EPAD = NS * 160 * EB
ROWS_PT = 624
ROWS_TL = N - NS * ROWS_PT
NPAD = N + CB
REP = 64
BLK = 1000
NBLKS = N // BLK

_mesh = plsc.VectorSubcoreMesh(core_axis_name="c", subcore_axis_name="s",
                               num_cores=NC, num_subcores=NS)
_f32 = jnp.float32
_i32 = jnp.int32


def _part_copy(sid, src, dst):
    pltpu.sync_copy(src.at[pl.ds(sid * ROWS_PT, ROWS_PT)],
                    dst.at[pl.ds(sid * ROWS_PT, ROWS_PT)])

    @pl.when(sid == NS - 1)
    def _():
        pltpu.sync_copy(src.at[pl.ds(NS * ROWS_PT, ROWS_TL)],
                        dst.at[pl.ds(NS * ROWS_PT, ROWS_TL)])


def _edge_loop(tab_hbm, edg_hbm, acc, rows0, rows1, e0, e1, g0, g1, s0, s1,
               sid, nblk):
    pltpu.sync_copy(edg_hbm.at[sid, 0], e0)
    pltpu.sync_copy(edg_hbm.at[sid, 1], e1)
    pltpu.async_copy(tab_hbm.at[e0.at[0]], rows0, g0)
    pltpu.async_copy(tab_hbm.at[e1.at[0]], rows1, g1)

    def step(t, carry):
        j = 2 * t
        pltpu.make_async_copy(tab_hbm.at[e0.at[0]], rows0, g0).wait()
        pltpu.sync_copy(rows0, acc.at[e0.at[1]], add=True)

        @pl.when(j + 2 < nblk)
        def _():
            pltpu.async_copy(edg_hbm.at[sid, j + 2], e0, s0)

        pltpu.make_async_copy(tab_hbm.at[e1.at[0]], rows1, g1).wait()
        pltpu.sync_copy(rows1, acc.at[e1.at[1]], add=True)

        @pl.when(j + 3 < nblk)
        def _():
            pltpu.async_copy(edg_hbm.at[sid, j + 3], e1, s1)

        @pl.when(j + 2 < nblk)
        def _():
            pltpu.make_async_copy(edg_hbm.at[sid, j + 2], e0, s0).wait()
            pltpu.async_copy(tab_hbm.at[e0.at[0]], rows0, g0)

        @pl.when(j + 3 < nblk)
        def _():
            pltpu.make_async_copy(edg_hbm.at[sid, j + 3], e1, s1).wait()
            pltpu.async_copy(tab_hbm.at[e1.at[0]], rows1, g1)
        return carry

    lax.fori_loop(0, (nblk + 1) // 2, step, 0)


def _gs_body(init_hbm, tab_hbm, edgA_hbm, edgB_hbm, out0, out1, rows0, rows1,
             e0, e1, acc, g0, g1, s0, s1):
    cid = lax.axis_index("c")
    sid = lax.axis_index("s")
    _part_copy(sid, init_hbm, acc)
    plsc.subcore_barrier()

    @pl.when(cid == 0)
    def _():
        _edge_loop(tab_hbm, edgA_hbm, acc, rows0, rows1, e0, e1, g0, g1,
                   s0, s1, sid, NBLK0)

    @pl.when(cid == 1)
    def _():
        _edge_loop(tab_hbm, edgB_hbm, acc, rows0, rows1, e0, e1, g0, g1,
                   s0, s1, sid, NBLK1)

    plsc.subcore_barrier()

    @pl.when(cid == 0)
    def _():
        _part_copy(sid, acc, out0)

    @pl.when(cid == 1)
    def _():
        _part_copy(sid, acc, out1)


_gs_call = pl.kernel(
    _gs_body,
    out_type=(jax.ShapeDtypeStruct((N, D), _f32),
              jax.ShapeDtypeStruct((N, D), _f32)),
    mesh=_mesh,
    scratch_types=[
        pltpu.VMEM((EB, D), _f32),
        pltpu.VMEM((EB, D), _f32),
        pltpu.VMEM((2, EB), _i32),
        pltpu.VMEM((2, EB), _i32),
        pltpu.VMEM_SHARED((NPAD, D), _f32),
        pltpu.SemaphoreType.DMA,
        pltpu.SemaphoreType.DMA,
        pltpu.SemaphoreType.DMA,
        pltpu.SemaphoreType.DMA,
    ],
)


def _h0_body(x0_ref, x1_ref, emb_ref, o_ref):
    cols = lax.broadcasted_iota(_i32, (BLK, 128), 1)
    oh = ((x0_ref[...] == cols).astype(_f32)
          + (x1_ref[...] == cols).astype(_f32))
    o_ref[...] = jnp.dot(oh, emb_ref[...], preferred_element_type=_f32,
                         precision=lax.Precision.HIGHEST)


_h0_call = pl.pallas_call(
    _h0_body,
    grid=(NBLKS,),
    in_specs=[
        pl.BlockSpec((BLK, 1), lambda i: (i, 0)),
        pl.BlockSpec((BLK, 1), lambda i: (i, 0)),
        pl.BlockSpec((128, D), lambda i: (0, 0)),
    ],
    out_specs=pl.BlockSpec((BLK, D), lambda i: (i, 0)),
    out_shape=jax.ShapeDtypeStruct((N, D), _f32),
)


def _c1_body(p0, p1, h, c0, c1, v, sv, w1, b1, w2, b2, h2o, stats):
    i = pl.program_id(0)
    cnt = c0[...] + c1[...]
    agg = (p0[...] + p1[...] - h[...] + sv[...]
           + jnp.dot(cnt, v[...], preferred_element_type=_f32,
                     precision=lax.Precision.HIGHEST))
    hmid = jnp.maximum(
        jnp.dot(agg, w1[...], preferred_element_type=_f32) + b1[...], 0.0)
    h2 = jnp.dot(hmid, w2[...], preferred_element_type=_f32) + b2[...]
    h2o[...] = h2
    st = jnp.concatenate([jnp.sum(h2, axis=0, keepdims=True),
                          jnp.sum(h2 * h2, axis=0, keepdims=True)], axis=0)

    @pl.when(i == 0)
    def _():
        stats[...] = st

    @pl.when(i > 0)
    def _():
        stats[...] = stats[...] + st


_c1_call = pl.pallas_call(
    _c1_body,
    grid=(NBLKS,),
    in_specs=[
        pl.BlockSpec((BLK, D), lambda i: (i, 0)),
        pl.BlockSpec((BLK, D), lambda i: (i, 0)),
        pl.BlockSpec((BLK, D), lambda i: (i, 0)),
        pl.BlockSpec((BLK, D), lambda i: (i, 0)),
        pl.BlockSpec((BLK, D), lambda i: (i, 0)),
        pl.BlockSpec((128, D), lambda i: (0, 0)),
        pl.BlockSpec((1, D), lambda i: (0, 0)),
        pl.BlockSpec((D, 2 * D), lambda i: (0, 0)),
        pl.BlockSpec((1, 2 * D), lambda i: (0, 0)),
        pl.BlockSpec((2 * D, D), lambda i: (0, 0)),
        pl.BlockSpec((1, D), lambda i: (0, 0)),
    ],
    out_specs=(pl.BlockSpec((BLK, D), lambda i: (i, 0)),
               pl.BlockSpec((2, D), lambda i: (0, 0))),
    out_shape=(jax.ShapeDtypeStruct((N, D), _f32),
               jax.ShapeDtypeStruct((2, D), _f32)),
)


def _c2_body(h2, stats, g, b, o, *, use_elu):
    mu = stats[0:1, :] * (1.0 / N)
    var = stats[1:2, :] * (1.0 / N) - mu * mu
    inv = 1.0 / jnp.sqrt(var + 1e-5)
    y = (h2[...] - mu) * (inv * g[...]) + b[...]
    if use_elu:
        y = jnp.where(y > 0, y, jnp.exp(y) - 1.0)
    o[...] = y


def _make_c2(use_elu):
    return pl.pallas_call(
        functools.partial(_c2_body, use_elu=use_elu),
        grid=(NBLKS,),
        in_specs=[
            pl.BlockSpec((BLK, D), lambda i: (i, 0)),
            pl.BlockSpec((2, D), lambda i: (0, 0)),
            pl.BlockSpec((1, D), lambda i: (0, 0)),
            pl.BlockSpec((1, D), lambda i: (0, 0)),
        ],
        out_specs=pl.BlockSpec((BLK, D), lambda i: (i, 0)),
        out_shape=jax.ShapeDtypeStruct((N, D), _f32),
    )


_c2_elu = _make_c2(True)
_c2_last = _make_c2(False)


def kernel(x, edge_index, edge_attr, x_emb, e1, e2, W1, b1, W2, b2, gamma,
           beta):
    x = x.astype(_i32)
    ei = edge_index.astype(_i32)
    ea = edge_attr.astype(_i32)
    e = ei.shape[1]
    npad = EPAD - e
    src = jnp.concatenate([ei[0], jnp.zeros((npad,), _i32)])
    dst = jnp.concatenate([ei[1], jnp.full((npad,), N, _i32)])
    combo = jnp.concatenate([ea[:, 0] * 3 + ea[:, 1], jnp.zeros((npad,), _i32)])
    combo = combo + CB * (jnp.arange(EPAD, dtype=_i32) % REP)

    def split(s_idx, d_idx):
        pairs = jnp.stack([s_idx.reshape(-1, EB), d_idx.reshape(-1, EB)],
                          axis=1)
        a = pairs[: NS * NBLK0].reshape(NS, NBLK0, 2, EB)
        b = pairs[NS * NBLK0:].reshape(NS, NBLK1, 2, EB)
        return a, b

    edgA, edgB = split(src, dst)
    cedgA, cedgB = split(combo, dst)
    emb_pad = jnp.zeros((128, D), _f32).at[: x_emb.shape[0]].set(x_emb)
    ohtab = jnp.tile(jnp.eye(CB, D, dtype=_f32), (REP, 1))
    zn = jnp.zeros((N, D), _f32)

    h = _h0_call(x[:, 0:1], x[:, 1:2], emb_pad)
    c0, c1 = _gs_call(zn, ohtab, cedgA, cedgB)

    combos = jnp.arange(9)
    for l in range(L):
        v = jnp.zeros((128, D), _f32).at[:9].set(
            e1[l][combos // 3] + e2[l][combos % 3])
        sv = (e1[l][4] + e2[l][0]).reshape(1, D)
        p0, p1 = _gs_call(h, h, edgA, edgB)
        h2, stats = _c1_call(p0, p1, h, c0, c1, v, sv, W1[l],
                             b1[l].reshape(1, -1), W2[l], b2[l].reshape(1, -1))
        c2 = _c2_elu if l < L - 1 else _c2_last
        h = c2(h2, stats, gamma[l].reshape(1, -1), beta[l].reshape(1, -1))
    return h


# speedup vs baseline: 1.2222x; 1.2222x over previous
import functools

import jax
import jax.numpy as jnp
from jax import lax
from jax.experimental import pallas as pl
from jax.experimental.pallas import tpu as pltpu
from jax.experimental.pallas import tpu_sc as plsc

N = 10000
D = 128
L = 3
CB = 16
NC = 2
NS = 16
NW = NC * NS
EB = 128
NBLK0 = 80
NBLK1 = 80
NBT = NBLK0 + NBLK1
EPAD = NS * 160 * EB
ROWS_PT = 624
ROWS_TL = N - NS * ROWS_PT
NPAD = N + CB
REP = 64
BLK = 1000
NBLKS = N // BLK

_mesh = plsc.VectorSubcoreMesh(core_axis_name="c", subcore_axis_name="s",
                               num_cores=NC, num_subcores=NS)
_f32 = jnp.float32
_i32 = jnp.int32


def _part_copy(sid, src, dst):
    pltpu.sync_copy(src.at[pl.ds(sid * ROWS_PT, ROWS_PT)],
                    dst.at[pl.ds(sid * ROWS_PT, ROWS_PT)])

    @pl.when(sid == NS - 1)
    def _():
        pltpu.sync_copy(src.at[pl.ds(NS * ROWS_PT, ROWS_TL)],
                        dst.at[pl.ds(NS * ROWS_PT, ROWS_TL)])


def _edge_loop(tab_hbm, edg_hbm, acc, rows0, rows1, e0, e1, g0, g1, s0, s1,
               sid, nblk):
    pltpu.sync_copy(edg_hbm.at[sid, 0], e0)
    pltpu.sync_copy(edg_hbm.at[sid, 1], e1)
    pltpu.async_copy(tab_hbm.at[e0.at[0]], rows0, g0)
    pltpu.async_copy(tab_hbm.at[e1.at[0]], rows1, g1)

    def step(t, carry):
        j = 2 * t
        pltpu.make_async_copy(tab_hbm.at[e0.at[0]], rows0, g0).wait()
        pltpu.sync_copy(rows0, acc.at[e0.at[1]], add=True)

        @pl.when(j + 2 < nblk)
        def _():
            pltpu.async_copy(edg_hbm.at[sid, j + 2], e0, s0)

        pltpu.make_async_copy(tab_hbm.at[e1.at[0]], rows1, g1).wait()
        pltpu.sync_copy(rows1, acc.at[e1.at[1]], add=True)

        @pl.when(j + 3 < nblk)
        def _():
            pltpu.async_copy(edg_hbm.at[sid, j + 3], e1, s1)

        @pl.when(j + 2 < nblk)
        def _():
            pltpu.make_async_copy(edg_hbm.at[sid, j + 2], e0, s0).wait()
            pltpu.async_copy(tab_hbm.at[e0.at[0]], rows0, g0)

        @pl.when(j + 3 < nblk)
        def _():
            pltpu.make_async_copy(edg_hbm.at[sid, j + 3], e1, s1).wait()
            pltpu.async_copy(tab_hbm.at[e1.at[0]], rows1, g1)
        return carry

    lax.fori_loop(0, (nblk + 1) // 2, step, 0)


def _gs_body(init_hbm, tab_hbm, edgA_hbm, edgB_hbm, out0, out1, rows0, rows1,
             e0, e1, acc, g0, g1, s0, s1):
    cid = lax.axis_index("c")
    sid = lax.axis_index("s")
    _part_copy(sid, init_hbm, acc)
    plsc.subcore_barrier()

    @pl.when(cid == 0)
    def _():
        _edge_loop(tab_hbm, edgA_hbm, acc, rows0, rows1, e0, e1, g0, g1,
                   s0, s1, sid, NBLK0)

    @pl.when(cid == 1)
    def _():
        _edge_loop(tab_hbm, edgB_hbm, acc, rows0, rows1, e0, e1, g0, g1,
                   s0, s1, sid, NBLK1)

    plsc.subcore_barrier()

    @pl.when(cid == 0)
    def _():
        _part_copy(sid, acc, out0)

    @pl.when(cid == 1)
    def _():
        _part_copy(sid, acc, out1)


_gs_call = pl.kernel(
    _gs_body,
    out_type=(jax.ShapeDtypeStruct((N, D), _f32),
              jax.ShapeDtypeStruct((N, D), _f32)),
    mesh=_mesh,
    scratch_types=[
        pltpu.VMEM((EB, D), _f32),
        pltpu.VMEM((EB, D), _f32),
        pltpu.VMEM((2, EB), _i32),
        pltpu.VMEM((2, EB), _i32),
        pltpu.VMEM_SHARED((NPAD, D), _f32),
        pltpu.SemaphoreType.DMA,
        pltpu.SemaphoreType.DMA,
        pltpu.SemaphoreType.DMA,
        pltpu.SemaphoreType.DMA,
    ],
)


def _h0_body(x0_ref, x1_ref, emb_ref, o_ref):
    cols = lax.broadcasted_iota(_i32, (BLK, 128), 1)
    oh = ((x0_ref[...] == cols).astype(_f32)
          + (x1_ref[...] == cols).astype(_f32))
    o_ref[...] = jnp.dot(oh, emb_ref[...], preferred_element_type=_f32,
                         precision=lax.Precision.HIGHEST)


_h0_call = pl.pallas_call(
    _h0_body,
    grid=(NBLKS,),
    in_specs=[
        pl.BlockSpec((BLK, 1), lambda i: (i, 0)),
        pl.BlockSpec((BLK, 1), lambda i: (i, 0)),
        pl.BlockSpec((128, D), lambda i: (0, 0)),
    ],
    out_specs=pl.BlockSpec((BLK, D), lambda i: (i, 0)),
    out_shape=jax.ShapeDtypeStruct((N, D), _f32),
)


def _c1_body(p0, p1, h, c0, c1, v, sv, w1, b1, w2, b2, h2o, stats):
    i = pl.program_id(0)
    cnt = c0[...] + c1[...]
    agg = (p0[...] + p1[...] - h[...] + sv[...]
           + jnp.dot(cnt, v[...], preferred_element_type=_f32,
                     precision=lax.Precision.HIGHEST))
    hmid = jnp.maximum(
        jnp.dot(agg, w1[...], preferred_element_type=_f32) + b1[...], 0.0)
    h2 = jnp.dot(hmid, w2[...], preferred_element_type=_f32) + b2[...]
    h2o[...] = h2
    st = jnp.concatenate([jnp.sum(h2, axis=0, keepdims=True),
                          jnp.sum(h2 * h2, axis=0, keepdims=True)], axis=0)

    @pl.when(i == 0)
    def _():
        stats[...] = st

    @pl.when(i > 0)
    def _():
        stats[...] = stats[...] + st


_c1_call = pl.pallas_call(
    _c1_body,
    grid=(NBLKS,),
    in_specs=[
        pl.BlockSpec((BLK, D), lambda i: (i, 0)),
        pl.BlockSpec((BLK, D), lambda i: (i, 0)),
        pl.BlockSpec((BLK, D), lambda i: (i, 0)),
        pl.BlockSpec((BLK, D), lambda i: (i, 0)),
        pl.BlockSpec((BLK, D), lambda i: (i, 0)),
        pl.BlockSpec((128, D), lambda i: (0, 0)),
        pl.BlockSpec((1, D), lambda i: (0, 0)),
        pl.BlockSpec((D, 2 * D), lambda i: (0, 0)),
        pl.BlockSpec((1, 2 * D), lambda i: (0, 0)),
        pl.BlockSpec((2 * D, D), lambda i: (0, 0)),
        pl.BlockSpec((1, D), lambda i: (0, 0)),
    ],
    out_specs=(pl.BlockSpec((BLK, D), lambda i: (i, 0)),
               pl.BlockSpec((2, D), lambda i: (0, 0))),
    out_shape=(jax.ShapeDtypeStruct((N, D), _f32),
               jax.ShapeDtypeStruct((2, D), _f32)),
)


def _c2_body(h2, stats, g, b, o, *, use_elu):
    mu = stats[0:1, :] * (1.0 / N)
    var = stats[1:2, :] * (1.0 / N) - mu * mu
    inv = 1.0 / jnp.sqrt(var + 1e-5)
    y = (h2[...] - mu) * (inv * g[...]) + b[...]
    if use_elu:
        y = jnp.where(y > 0, y, jnp.exp(y) - 1.0)
    o[...] = y


def _make_c2(use_elu):
    return pl.pallas_call(
        functools.partial(_c2_body, use_elu=use_elu),
        grid=(NBLKS,),
        in_specs=[
            pl.BlockSpec((BLK, D), lambda i: (i, 0)),
            pl.BlockSpec((2, D), lambda i: (0, 0)),
            pl.BlockSpec((1, D), lambda i: (0, 0)),
            pl.BlockSpec((1, D), lambda i: (0, 0)),
        ],
        out_specs=pl.BlockSpec((BLK, D), lambda i: (i, 0)),
        out_shape=jax.ShapeDtypeStruct((N, D), _f32),
    )


_c2_elu = _make_c2(True)
_c2_last = _make_c2(False)


def kernel(x, edge_index, edge_attr, x_emb, e1, e2, W1, b1, W2, b2, gamma,
           beta):
    x = x.astype(_i32)
    ei = edge_index.astype(_i32)
    ea = edge_attr.astype(_i32)
    e = ei.shape[1]
    npad = EPAD - e
    src = jnp.concatenate([ei[0], jnp.zeros((npad,), _i32)])
    dst = jnp.concatenate([ei[1], jnp.full((npad,), N, _i32)])
    combo = jnp.concatenate([ea[:, 0] * 3 + ea[:, 1], jnp.zeros((npad,), _i32)])
    combo = combo + CB * (jnp.arange(EPAD, dtype=_i32) % REP)

    def split(s_idx, d_idx):
        pairs = jnp.stack([s_idx.reshape(-1, EB), d_idx.reshape(-1, EB)],
                          axis=1)
        a = pairs[: NS * NBLK0].reshape(NS, NBLK0, 2, EB)
        b = pairs[NS * NBLK0:].reshape(NS, NBLK1, 2, EB)
        return a, b

    edgA, edgB = split(src, dst)
    cedgA, cedgB = split(combo, dst)
    emb_pad = jnp.zeros((128, D), _f32).at[: x_emb.shape[0]].set(x_emb)
    ohtab = jnp.tile(jnp.eye(CB, D, dtype=_f32), (REP, 1))
    zn = jnp.zeros((N, D), _f32)

    h = _h0_call(x[:, 0:1], x[:, 1:2], emb_pad)
    c0, c1 = _gs_call(zn, ohtab, cedgA, cedgB)

    combos = jnp.arange(9)
    for l in range(L):
        v = jnp.zeros((128, D), _f32).at[:9].set(
            e1[l][combos // 3] + e2[l][combos % 3])
        sv = (e1[l][4] + e2[l][0]).reshape(1, D)
        p0, p1 = _gs_call(h, h, edgA, edgB)
        h2, stats = _c1_call(p0, p1, h, c0, c1, v, sv, W1[l],
                             b1[l].reshape(1, -1), W2[l], b2[l].reshape(1, -1))
        c2 = _c2_elu if l < L - 1 else _c2_last
        h = c2(h2, stats, gamma[l].reshape(1, -1), beta[l].reshape(1, -1))
    return h


# speedup vs baseline: 1.3370x; 1.0939x over previous
import functools

import jax
import jax.numpy as jnp
from jax import lax
from jax.experimental import pallas as pl
from jax.experimental.pallas import tpu as pltpu
from jax.experimental.pallas import tpu_sc as plsc

N = 10000
D = 128
L = 3
CB = 16
NC = 2
NS = 16
NW = NC * NS
EB = 128
NBLK = 80
EPAD = NW * NBLK * EB
ROWS_PT = 624
ROWS_TL = N - NS * ROWS_PT
NPAD = N + CB
REP = 64
BLK = 1000
NBLKS = N // BLK

_mesh = plsc.VectorSubcoreMesh(core_axis_name="c", subcore_axis_name="s",
                               num_cores=NC, num_subcores=NS)
_f32 = jnp.float32
_i32 = jnp.int32


def _part_copy(sid, src, dst):
    pltpu.sync_copy(src.at[pl.ds(sid * ROWS_PT, ROWS_PT)],
                    dst.at[pl.ds(sid * ROWS_PT, ROWS_PT)])

    @pl.when(sid == NS - 1)
    def _():
        pltpu.sync_copy(src.at[pl.ds(NS * ROWS_PT, ROWS_TL)],
                        dst.at[pl.ds(NS * ROWS_PT, ROWS_TL)])


def _edge_loop(tab_hbm, src_hbm, dst_hbm, acc, idx_s, rows0, rows1, d0, d1,
               g0, g1, s0, s1, wid):
    pltpu.sync_copy(src_hbm.at[wid], idx_s)
    pltpu.async_copy(tab_hbm.at[idx_s.at[0]], rows0, g0)
    pltpu.async_copy(dst_hbm.at[wid, 0], d0, s0)

    def step(t, carry):
        j = 2 * t
        pltpu.async_copy(tab_hbm.at[idx_s.at[j + 1]], rows1, g1)
        pltpu.async_copy(dst_hbm.at[wid, j + 1], d1, s1)
        pltpu.make_async_copy(tab_hbm.at[idx_s.at[j]], rows0, g0).wait()
        pltpu.make_async_copy(dst_hbm.at[wid, j], d0, s0).wait()
        pltpu.sync_copy(rows0, acc.at[d0], add=True)

        @pl.when(j + 2 < NBLK)
        def _():
            pltpu.async_copy(tab_hbm.at[idx_s.at[j + 2]], rows0, g0)
            pltpu.async_copy(dst_hbm.at[wid, j + 2], d0, s0)

        pltpu.make_async_copy(tab_hbm.at[idx_s.at[j + 1]], rows1, g1).wait()
        pltpu.make_async_copy(dst_hbm.at[wid, j + 1], d1, s1).wait()
        pltpu.sync_copy(rows1, acc.at[d1], add=True)
        return carry

    lax.fori_loop(0, NBLK // 2, step, 0)


def _writeout(cid, sid, acc, out0, out1):
    @pl.when(cid == 0)
    def _():
        _part_copy(sid, acc, out0)

    @pl.when(cid == 1)
    def _():
        _part_copy(sid, acc, out1)


def _gs_body(init_hbm, tab_hbm, src_hbm, dst_hbm, out0, out1, idx_s, rows0,
             rows1, d0, d1, acc, g0, g1, s0, s1):
    cid = lax.axis_index("c")
    sid = lax.axis_index("s")
    wid = cid * NS + sid
    _part_copy(sid, init_hbm, acc)
    plsc.subcore_barrier()
    _edge_loop(tab_hbm, src_hbm, dst_hbm, acc, idx_s, rows0, rows1, d0, d1,
               g0, g1, s0, s1, wid)
    plsc.subcore_barrier()
    _writeout(cid, sid, acc, out0, out1)


_GS_SCRATCH = [
    pltpu.VMEM((NBLK, EB), _i32),
    pltpu.VMEM((EB, D), _f32),
    pltpu.VMEM((EB, D), _f32),
    pltpu.VMEM((EB,), _i32),
    pltpu.VMEM((EB,), _i32),
    pltpu.VMEM_SHARED((NPAD, D), _f32),
    pltpu.SemaphoreType.DMA,
    pltpu.SemaphoreType.DMA,
    pltpu.SemaphoreType.DMA,
    pltpu.SemaphoreType.DMA,
]

_gs_call = pl.kernel(
    _gs_body,
    out_type=(jax.ShapeDtypeStruct((N, D), _f32),
              jax.ShapeDtypeStruct((N, D), _f32)),
    mesh=_mesh,
    scratch_types=_GS_SCRATCH,
)


def _gsd_body(init_hbm, tab_hbm, src_hbm, dst_hbm, zn_hbm, ohtab_hbm,
              csrc_hbm, out0, out1, cout0, cout1, idx_s, rows0, rows1, d0,
              d1, acc, g0, g1, s0, s1):
    cid = lax.axis_index("c")
    sid = lax.axis_index("s")
    wid = cid * NS + sid
    _part_copy(sid, init_hbm, acc)
    plsc.subcore_barrier()
    _edge_loop(tab_hbm, src_hbm, dst_hbm, acc, idx_s, rows0, rows1, d0, d1,
               g0, g1, s0, s1, wid)
    plsc.subcore_barrier()
    _writeout(cid, sid, acc, out0, out1)
    _part_copy(sid, zn_hbm, acc)
    plsc.subcore_barrier()
    _edge_loop(ohtab_hbm, csrc_hbm, dst_hbm, acc, idx_s, rows0, rows1, d0,
               d1, g0, g1, s0, s1, wid)
    plsc.subcore_barrier()
    _writeout(cid, sid, acc, cout0, cout1)


_gsd_call = pl.kernel(
    _gsd_body,
    out_type=(jax.ShapeDtypeStruct((N, D), _f32),
              jax.ShapeDtypeStruct((N, D), _f32),
              jax.ShapeDtypeStruct((N, D), _f32),
              jax.ShapeDtypeStruct((N, D), _f32)),
    mesh=_mesh,
    scratch_types=_GS_SCRATCH,
)


def _h0_body(x0_ref, x1_ref, emb_ref, o_ref):
    cols = lax.broadcasted_iota(_i32, (BLK, 128), 1)
    oh = ((x0_ref[...] == cols).astype(_f32)
          + (x1_ref[...] == cols).astype(_f32))
    o_ref[...] = jnp.dot(oh, emb_ref[...], preferred_element_type=_f32,
                         precision=lax.Precision.HIGHEST)


_h0_call = pl.pallas_call(
    _h0_body,
    grid=(NBLKS,),
    in_specs=[
        pl.BlockSpec((BLK, 1), lambda i: (i, 0)),
        pl.BlockSpec((BLK, 1), lambda i: (i, 0)),
        pl.BlockSpec((128, D), lambda i: (0, 0)),
    ],
    out_specs=pl.BlockSpec((BLK, D), lambda i: (i, 0)),
    out_shape=jax.ShapeDtypeStruct((N, D), _f32),
)


def _c1_body(p0, p1, h, c0, c1, v, sv, w1, b1, w2, b2, h2o, stats):
    i = pl.program_id(0)
    cnt = c0[...] + c1[...]
    agg = (p0[...] + p1[...] - h[...] + sv[...]
           + jnp.dot(cnt, v[...], preferred_element_type=_f32,
                     precision=lax.Precision.HIGHEST))
    hmid = jnp.maximum(
        jnp.dot(agg, w1[...], preferred_element_type=_f32) + b1[...], 0.0)
    h2 = jnp.dot(hmid, w2[...], preferred_element_type=_f32) + b2[...]
    h2o[...] = h2
    st = jnp.concatenate([jnp.sum(h2, axis=0, keepdims=True),
                          jnp.sum(h2 * h2, axis=0, keepdims=True)], axis=0)

    @pl.when(i == 0)
    def _():
        stats[...] = st

    @pl.when(i > 0)
    def _():
        stats[...] = stats[...] + st


_c1_call = pl.pallas_call(
    _c1_body,
    grid=(NBLKS,),
    in_specs=[
        pl.BlockSpec((BLK, D), lambda i: (i, 0)),
        pl.BlockSpec((BLK, D), lambda i: (i, 0)),
        pl.BlockSpec((BLK, D), lambda i: (i, 0)),
        pl.BlockSpec((BLK, D), lambda i: (i, 0)),
        pl.BlockSpec((BLK, D), lambda i: (i, 0)),
        pl.BlockSpec((128, D), lambda i: (0, 0)),
        pl.BlockSpec((1, D), lambda i: (0, 0)),
        pl.BlockSpec((D, 2 * D), lambda i: (0, 0)),
        pl.BlockSpec((1, 2 * D), lambda i: (0, 0)),
        pl.BlockSpec((2 * D, D), lambda i: (0, 0)),
        pl.BlockSpec((1, D), lambda i: (0, 0)),
    ],
    out_specs=(pl.BlockSpec((BLK, D), lambda i: (i, 0)),
               pl.BlockSpec((2, D), lambda i: (0, 0))),
    out_shape=(jax.ShapeDtypeStruct((N, D), _f32),
               jax.ShapeDtypeStruct((2, D), _f32)),
)


def _c2_body(h2, stats, g, b, o, *, use_elu):
    mu = stats[0:1, :] * (1.0 / N)
    var = stats[1:2, :] * (1.0 / N) - mu * mu
    inv = 1.0 / jnp.sqrt(var + 1e-5)
    y = (h2[...] - mu) * (inv * g[...]) + b[...]
    if use_elu:
        y = jnp.where(y > 0, y, jnp.exp(y) - 1.0)
    o[...] = y


def _make_c2(use_elu):
    return pl.pallas_call(
        functools.partial(_c2_body, use_elu=use_elu),
        grid=(NBLKS,),
        in_specs=[
            pl.BlockSpec((BLK, D), lambda i: (i, 0)),
            pl.BlockSpec((2, D), lambda i: (0, 0)),
            pl.BlockSpec((1, D), lambda i: (0, 0)),
            pl.BlockSpec((1, D), lambda i: (0, 0)),
        ],
        out_specs=pl.BlockSpec((BLK, D), lambda i: (i, 0)),
        out_shape=jax.ShapeDtypeStruct((N, D), _f32),
    )


_c2_elu = _make_c2(True)
_c2_last = _make_c2(False)


def kernel(x, edge_index, edge_attr, x_emb, e1, e2, W1, b1, W2, b2, gamma,
           beta):
    x = x.astype(_i32)
    ei = edge_index.astype(_i32)
    ea = edge_attr.astype(_i32)
    e = ei.shape[1]
    npad = EPAD - e
    src = jnp.concatenate([ei[0], jnp.zeros((npad,), _i32)])
    dst = jnp.concatenate([ei[1], jnp.full((npad,), N, _i32)])
    combo = jnp.concatenate([ea[:, 0] * 3 + ea[:, 1], jnp.zeros((npad,), _i32)])
    combo = combo + CB * (jnp.arange(EPAD, dtype=_i32) % REP)
    src = src.reshape(NW, NBLK, EB)
    dst = dst.reshape(NW, NBLK, EB)
    combo = combo.reshape(NW, NBLK, EB)
    emb_pad = jnp.zeros((128, D), _f32).at[: x_emb.shape[0]].set(x_emb)
    ohtab = jnp.tile(jnp.eye(CB, D, dtype=_f32), (REP, 1))
    zn = jnp.zeros((N, D), _f32)

    h = _h0_call(x[:, 0:1], x[:, 1:2], emb_pad)

    combos = jnp.arange(9)
    c0 = c1 = None
    for l in range(L):
        v = jnp.zeros((128, D), _f32).at[:9].set(
            e1[l][combos // 3] + e2[l][combos % 3])
        sv = (e1[l][4] + e2[l][0]).reshape(1, D)
        if l == 0:
            p0, p1, c0, c1 = _gsd_call(h, h, src, dst, zn, ohtab, combo)
        else:
            p0, p1 = _gs_call(h, h, src, dst)
        h2, stats = _c1_call(p0, p1, h, c0, c1, v, sv, W1[l],
                             b1[l].reshape(1, -1), W2[l], b2[l].reshape(1, -1))
        c2 = _c2_elu if l < L - 1 else _c2_last
        h = c2(h2, stats, gamma[l].reshape(1, -1), beta[l].reshape(1, -1))
    return h


# speedup vs baseline: 1.4221x; 1.0636x over previous
import functools

import jax
import jax.numpy as jnp
from jax import lax
from jax.experimental import pallas as pl
from jax.experimental.pallas import tpu as pltpu
from jax.experimental.pallas import tpu_sc as plsc

N = 10000
D = 128
L = 3
CB = 16
NC = 2
NS = 16
NW = NC * NS
EB = 128
NBLK = 80
EPAD = NW * NBLK * EB
ROWS_PT = 624
ROWS_TL = N - NS * ROWS_PT
NPAD = N + CB
REP = 64
BLK = 1000
NBLKS = N // BLK

_mesh = plsc.VectorSubcoreMesh(core_axis_name="c", subcore_axis_name="s",
                               num_cores=NC, num_subcores=NS)
_f32 = jnp.float32
_i32 = jnp.int32


def _part_copy(sid, src, dst):
    pltpu.sync_copy(src.at[pl.ds(sid * ROWS_PT, ROWS_PT)],
                    dst.at[pl.ds(sid * ROWS_PT, ROWS_PT)])

    @pl.when(sid == NS - 1)
    def _():
        pltpu.sync_copy(src.at[pl.ds(NS * ROWS_PT, ROWS_TL)],
                        dst.at[pl.ds(NS * ROWS_PT, ROWS_TL)])


def _edge_loop(tab_hbm, src_hbm, dst_hbm, acc, idx_s, rows0, rows1, d0, d1,
               g0, g1, s0, s1, wid):
    pltpu.sync_copy(src_hbm.at[wid], idx_s)
    pltpu.async_copy(tab_hbm.at[idx_s.at[0]], rows0, g0)
    pltpu.async_copy(dst_hbm.at[wid, 0], d0, s0)

    def step(t, carry):
        j = 2 * t
        pltpu.async_copy(tab_hbm.at[idx_s.at[j + 1]], rows1, g1)
        pltpu.async_copy(dst_hbm.at[wid, j + 1], d1, s1)
        pltpu.make_async_copy(tab_hbm.at[idx_s.at[j]], rows0, g0).wait()
        pltpu.make_async_copy(dst_hbm.at[wid, j], d0, s0).wait()
        pltpu.sync_copy(rows0, acc.at[d0], add=True)

        @pl.when(j + 2 < NBLK)
        def _():
            pltpu.async_copy(tab_hbm.at[idx_s.at[j + 2]], rows0, g0)
            pltpu.async_copy(dst_hbm.at[wid, j + 2], d0, s0)

        pltpu.make_async_copy(tab_hbm.at[idx_s.at[j + 1]], rows1, g1).wait()
        pltpu.make_async_copy(dst_hbm.at[wid, j + 1], d1, s1).wait()
        pltpu.sync_copy(rows1, acc.at[d1], add=True)
        return carry

    lax.fori_loop(0, NBLK // 2, step, 0)


def _writeout(cid, sid, acc, out0, out1):
    @pl.when(cid == 0)
    def _():
        _part_copy(sid, acc, out0)

    @pl.when(cid == 1)
    def _():
        _part_copy(sid, acc, out1)


def _gs_body(init_hbm, tab_hbm, src_hbm, dst_hbm, out0, out1, idx_s, rows0,
             rows1, d0, d1, acc, g0, g1, s0, s1):
    cid = lax.axis_index("c")
    sid = lax.axis_index("s")
    wid = cid * NS + sid
    _part_copy(sid, init_hbm, acc)
    plsc.subcore_barrier()
    _edge_loop(tab_hbm, src_hbm, dst_hbm, acc, idx_s, rows0, rows1, d0, d1,
               g0, g1, s0, s1, wid)
    plsc.subcore_barrier()
    _writeout(cid, sid, acc, out0, out1)


_GS_SCRATCH = [
    pltpu.VMEM((NBLK, EB), _i32),
    pltpu.VMEM((EB, D), _f32),
    pltpu.VMEM((EB, D), _f32),
    pltpu.VMEM((EB,), _i32),
    pltpu.VMEM((EB,), _i32),
    pltpu.VMEM_SHARED((NPAD, D), _f32),
    pltpu.SemaphoreType.DMA,
    pltpu.SemaphoreType.DMA,
    pltpu.SemaphoreType.DMA,
    pltpu.SemaphoreType.DMA,
]

_gs_call = pl.kernel(
    _gs_body,
    out_type=(jax.ShapeDtypeStruct((N, D), _f32),
              jax.ShapeDtypeStruct((N, D), _f32)),
    mesh=_mesh,
    scratch_types=_GS_SCRATCH,
)


def _h0_body(x0_ref, x1_ref, emb_ref, o_ref):
    cols = lax.broadcasted_iota(_i32, (BLK, 128), 1)
    oh = ((x0_ref[...] == cols).astype(_f32)
          + (x1_ref[...] == cols).astype(_f32))
    o_ref[...] = jnp.dot(oh, emb_ref[...], preferred_element_type=_f32,
                         precision=lax.Precision.HIGHEST)


_h0_call = pl.pallas_call(
    _h0_body,
    grid=(NBLKS,),
    in_specs=[
        pl.BlockSpec((BLK, 1), lambda i: (i, 0)),
        pl.BlockSpec((BLK, 1), lambda i: (i, 0)),
        pl.BlockSpec((128, D), lambda i: (0, 0)),
    ],
    out_specs=pl.BlockSpec((BLK, D), lambda i: (i, 0)),
    out_shape=jax.ShapeDtypeStruct((N, D), _f32),
)


def _c1_body(p0, p1, h, c0, c1, v, sv, w1, b1, w2, b2, h2o, stats):
    i = pl.program_id(0)
    cnt = c0[...] + c1[...]
    agg = (p0[...] + p1[...] - h[...] + sv[...]
           + jnp.dot(cnt, v[...], preferred_element_type=_f32,
                     precision=lax.Precision.HIGHEST))
    hmid = jnp.maximum(
        jnp.dot(agg, w1[...], preferred_element_type=_f32) + b1[...], 0.0)
    h2 = jnp.dot(hmid, w2[...], preferred_element_type=_f32) + b2[...]
    h2o[...] = h2
    st = jnp.concatenate([jnp.sum(h2, axis=0, keepdims=True),
                          jnp.sum(h2 * h2, axis=0, keepdims=True)], axis=0)

    @pl.when(i == 0)
    def _():
        stats[...] = st

    @pl.when(i > 0)
    def _():
        stats[...] = stats[...] + st


_c1_call = pl.pallas_call(
    _c1_body,
    grid=(NBLKS,),
    in_specs=[
        pl.BlockSpec((BLK, D), lambda i: (i, 0)),
        pl.BlockSpec((BLK, D), lambda i: (i, 0)),
        pl.BlockSpec((BLK, D), lambda i: (i, 0)),
        pl.BlockSpec((BLK, D), lambda i: (i, 0)),
        pl.BlockSpec((BLK, D), lambda i: (i, 0)),
        pl.BlockSpec((128, D), lambda i: (0, 0)),
        pl.BlockSpec((1, D), lambda i: (0, 0)),
        pl.BlockSpec((D, 2 * D), lambda i: (0, 0)),
        pl.BlockSpec((1, 2 * D), lambda i: (0, 0)),
        pl.BlockSpec((2 * D, D), lambda i: (0, 0)),
        pl.BlockSpec((1, D), lambda i: (0, 0)),
    ],
    out_specs=(pl.BlockSpec((BLK, D), lambda i: (i, 0)),
               pl.BlockSpec((2, D), lambda i: (0, 0))),
    out_shape=(jax.ShapeDtypeStruct((N, D), _f32),
               jax.ShapeDtypeStruct((2, D), _f32)),
)


def _c2_body(h2, stats, g, b, o, *, use_elu):
    mu = stats[0:1, :] * (1.0 / N)
    var = stats[1:2, :] * (1.0 / N) - mu * mu
    inv = 1.0 / jnp.sqrt(var + 1e-5)
    y = (h2[...] - mu) * (inv * g[...]) + b[...]
    if use_elu:
        y = jnp.where(y > 0, y, jnp.exp(y) - 1.0)
    o[...] = y


def _make_c2(use_elu):
    return pl.pallas_call(
        functools.partial(_c2_body, use_elu=use_elu),
        grid=(NBLKS,),
        in_specs=[
            pl.BlockSpec((BLK, D), lambda i: (i, 0)),
            pl.BlockSpec((2, D), lambda i: (0, 0)),
            pl.BlockSpec((1, D), lambda i: (0, 0)),
            pl.BlockSpec((1, D), lambda i: (0, 0)),
        ],
        out_specs=pl.BlockSpec((BLK, D), lambda i: (i, 0)),
        out_shape=jax.ShapeDtypeStruct((N, D), _f32),
    )


_c2_elu = _make_c2(True)
_c2_last = _make_c2(False)


def kernel(x, edge_index, edge_attr, x_emb, e1, e2, W1, b1, W2, b2, gamma,
           beta):
    x = x.astype(_i32)
    ei = edge_index.astype(_i32)
    ea = edge_attr.astype(_i32)
    e = ei.shape[1]
    npad = EPAD - e
    src = jnp.concatenate([ei[0], jnp.zeros((npad,), _i32)])
    dst = jnp.concatenate([ei[1], jnp.full((npad,), N, _i32)])
    combo = jnp.concatenate([ea[:, 0] * 3 + ea[:, 1], jnp.zeros((npad,), _i32)])
    combo = combo + CB * (jnp.arange(EPAD, dtype=_i32) % REP)
    src = src.reshape(NW, NBLK, EB)
    dst = dst.reshape(NW, NBLK, EB)
    combo = combo.reshape(NW, NBLK, EB)
    emb_pad = jnp.zeros((128, D), _f32).at[: x_emb.shape[0]].set(x_emb)
    ohtab = jnp.tile(jnp.eye(CB, D, dtype=_f32), (REP, 1))
    zn = jnp.zeros((N, D), _f32)

    h = _h0_call(x[:, 0:1], x[:, 1:2], emb_pad)

    c0, c1 = _gs_call(zn, ohtab, combo, dst)

    combos = jnp.arange(9)
    for l in range(L):
        v = jnp.zeros((128, D), _f32).at[:9].set(
            e1[l][combos // 3] + e2[l][combos % 3])
        sv = (e1[l][4] + e2[l][0]).reshape(1, D)
        p0, p1 = _gs_call(h, h, src, dst)
        h2, stats = _c1_call(p0, p1, h, c0, c1, v, sv, W1[l],
                             b1[l].reshape(1, -1), W2[l], b2[l].reshape(1, -1))
        c2 = _c2_elu if l < L - 1 else _c2_last
        h = c2(h2, stats, gamma[l].reshape(1, -1), beta[l].reshape(1, -1))
    return h
